# baseline (speedup 1.0000x reference)
import jax
import jax.numpy as jnp
from jax.experimental import pallas as pl
from jax.experimental.pallas import tpu as pltpu

_N = 5000
_B = 128
_NB = 40
_P = _NB * _B
_PHASE = 10
_IOU_T = 0.5


def _nms_kernel(x1w, y1w, x2w, y2w, keep_out, arear, keeprow):
    keep_out[...] = jnp.zeros((_NB, _B), jnp.float32)
    keeprow[...] = jnp.zeros((1, _P), jnp.float32)
    arear[...] = (x2w[...] - x1w[...]) * (y2w[...] - y1w[...])

    rloc = jax.lax.broadcasted_iota(jnp.int32, (_B, _B), 0)
    cloc = jax.lax.broadcasted_iota(jnp.int32, (_B, _B), 1)

    def block_step(i, strip_len):
        bx1 = x1w[0:1, pl.ds(i * _B, _B)]
        by1 = y1w[0:1, pl.ds(i * _B, _B)]
        bx2 = x2w[0:1, pl.ds(i * _B, _B)]
        by2 = y2w[0:1, pl.ds(i * _B, _B)]
        barea = (bx2 - bx1) * (by2 - by1)
        rx1 = jnp.transpose(bx1, (1, 0))
        ry1 = jnp.transpose(by1, (1, 0))
        rx2 = jnp.transpose(bx2, (1, 0))
        ry2 = jnp.transpose(by2, (1, 0))
        rarea = jnp.transpose(barea, (1, 0))

        sx1 = x1w[0:1, 0:strip_len]
        sy1 = y1w[0:1, 0:strip_len]
        sx2 = x2w[0:1, 0:strip_len]
        sy2 = y2w[0:1, 0:strip_len]
        sarea = arear[0:1, 0:strip_len]
        jx1 = jnp.maximum(rx1, sx1)
        jy1 = jnp.maximum(ry1, sy1)
        jx2 = jnp.minimum(rx2, sx2)
        jy2 = jnp.minimum(ry2, sy2)
        jw = jnp.maximum(jx2 - jx1, 0.0)
        jh = jnp.maximum(jy2 - jy1, 0.0)
        jinter = jw * jh
        jiou = jinter / (sarea + rarea - jinter + 1e-9)
        over = jiou > _IOU_T

        kmask = jnp.where(over, keeprow[0:1, pl.ds(0, strip_len)], 0.0)
        sup_col = jnp.max(kmask, axis=1, keepdims=True)
        free = jnp.transpose((sup_col == 0.0).astype(jnp.float32), (1, 0))

        tx1 = jnp.maximum(rx1, bx1)
        ty1 = jnp.maximum(ry1, by1)
        tx2 = jnp.minimum(rx2, bx2)
        ty2 = jnp.minimum(ry2, by2)
        tw = jnp.maximum(tx2 - tx1, 0.0)
        th = jnp.maximum(ty2 - ty1, 0.0)
        tinter = tw * th
        tiou = tinter / (rarea + barea - tinter + 1e-9)
        tf = jnp.where((tiou > _IOU_T) & (rloc < cloc), 1.0, 0.0)

        def fcond(c):
            return c[2]

        def fbody(c):
            k, _, _ = c
            s = jnp.dot(k, tf, preferred_element_type=jnp.float32)
            k2 = free * (s == 0.0).astype(jnp.float32)
            return k2, k, jnp.any(k2 != k)

        k0 = free
        s0 = jnp.dot(k0, tf, preferred_element_type=jnp.float32)
        k1 = free * (s0 == 0.0).astype(jnp.float32)
        k, _, _ = jax.lax.while_loop(fcond, fbody, (k1, k0, jnp.any(k1 != k0)))

        keep_out[pl.ds(i, 1), :] = k
        keeprow[0:1, pl.ds(i * _B, _B)] = k

    for p in range(_NB // _PHASE):
        strip_len = (p + 1) * _PHASE * _B

        def phase_body(i, carry, strip_len=strip_len):
            block_step(i, strip_len)
            return carry

        jax.lax.fori_loop(p * _PHASE, (p + 1) * _PHASE, phase_body, 0,
                          unroll=False)


def kernel(boxes, scores):
    pad = _P - _N
    neg_scores = jnp.concatenate(
        [-scores, jnp.full((pad,), jnp.inf, dtype=jnp.float32)])
    boxes_p = jnp.concatenate(
        [boxes, jnp.zeros((pad, 4), dtype=jnp.float32)], axis=0)
    idx = jnp.arange(_P, dtype=jnp.int32)

    _, sx1, sy1, sx2, sy2, sidx = jax.lax.sort(
        (neg_scores, boxes_p[:, 0], boxes_p[:, 1], boxes_p[:, 2],
         boxes_p[:, 3], idx),
        num_keys=1, is_stable=True)

    comps_w = [c.reshape(1, _P) for c in (sx1, sy1, sx2, sy2)]

    keep_sorted = pl.pallas_call(
        _nms_kernel,
        out_shape=jax.ShapeDtypeStruct((_NB, _B), jnp.float32),
        scratch_shapes=[pltpu.VMEM((1, _P), jnp.float32),
                        pltpu.VMEM((1, _P), jnp.float32)],
    )(*comps_w)

    packed = sidx * 2 + (keep_sorted.reshape(_P) > 0.0).astype(jnp.int32)
    (packed_sorted,) = jax.lax.sort((packed,), num_keys=1, is_stable=False)
    keep = (packed_sorted % 2 == 1)[:_N]
    masked = scores * keep.astype(scores.dtype)
    return masked, keep


# speedup vs baseline: 1.0612x; 1.0612x over previous
import jax
import jax.numpy as jnp
from jax.experimental import pallas as pl
from jax.experimental.pallas import tpu as pltpu

_N = 5000
_B = 128
_NB = 40
_P = _NB * _B
_PHASE = 5
_IOU_T = 0.5


def _nms_kernel(x1w, y1w, x2w, y2w, keep_out, arear, keeprow):
    keep_out[...] = jnp.zeros((_NB, _B), jnp.float32)
    keeprow[...] = jnp.zeros((1, _P), jnp.float32)
    arear[...] = (x2w[...] - x1w[...]) * (y2w[...] - y1w[...])

    rloc = jax.lax.broadcasted_iota(jnp.int32, (_B, _B), 0)
    cloc = jax.lax.broadcasted_iota(jnp.int32, (_B, _B), 1)

    def block_step(i, strip_len):
        bx1 = x1w[0:1, pl.ds(i * _B, _B)]
        by1 = y1w[0:1, pl.ds(i * _B, _B)]
        bx2 = x2w[0:1, pl.ds(i * _B, _B)]
        by2 = y2w[0:1, pl.ds(i * _B, _B)]
        barea = (bx2 - bx1) * (by2 - by1)
        rx1 = jnp.transpose(bx1, (1, 0))
        ry1 = jnp.transpose(by1, (1, 0))
        rx2 = jnp.transpose(bx2, (1, 0))
        ry2 = jnp.transpose(by2, (1, 0))
        rarea = jnp.transpose(barea, (1, 0))

        sx1 = x1w[0:1, 0:strip_len]
        sy1 = y1w[0:1, 0:strip_len]
        sx2 = x2w[0:1, 0:strip_len]
        sy2 = y2w[0:1, 0:strip_len]
        sarea = arear[0:1, 0:strip_len]
        jx1 = jnp.maximum(rx1, sx1)
        jy1 = jnp.maximum(ry1, sy1)
        jx2 = jnp.minimum(rx2, sx2)
        jy2 = jnp.minimum(ry2, sy2)
        jw = jnp.maximum(jx2 - jx1, 0.0)
        jh = jnp.maximum(jy2 - jy1, 0.0)
        jinter = jw * jh
        jiou = jinter / (sarea + rarea - jinter + 1e-9)
        over = jiou > _IOU_T

        kmask = jnp.where(over, keeprow[0:1, pl.ds(0, strip_len)], 0.0)
        sup_col = jnp.max(kmask, axis=1, keepdims=True)
        free = jnp.transpose((sup_col == 0.0).astype(jnp.float32), (1, 0))

        tx1 = jnp.maximum(rx1, bx1)
        ty1 = jnp.maximum(ry1, by1)
        tx2 = jnp.minimum(rx2, bx2)
        ty2 = jnp.minimum(ry2, by2)
        tw = jnp.maximum(tx2 - tx1, 0.0)
        th = jnp.maximum(ty2 - ty1, 0.0)
        tinter = tw * th
        tiou = tinter / (rarea + barea - tinter + 1e-9)
        tf = jnp.where((tiou > _IOU_T) & (rloc < cloc), 1.0, 0.0)

        def fcond(c):
            return c[2]

        def fbody(c):
            k, _, _ = c
            s = jnp.dot(k, tf, preferred_element_type=jnp.float32)
            k2 = free * (s == 0.0).astype(jnp.float32)
            return k2, k, jnp.any(k2 != k)

        k0 = free
        s0 = jnp.dot(k0, tf, preferred_element_type=jnp.float32)
        k1 = free * (s0 == 0.0).astype(jnp.float32)
        k, _, _ = jax.lax.while_loop(fcond, fbody, (k1, k0, jnp.any(k1 != k0)))

        keep_out[pl.ds(i, 1), :] = k
        keeprow[0:1, pl.ds(i * _B, _B)] = k

    for p in range(_NB // _PHASE):
        strip_len = (p + 1) * _PHASE * _B

        def phase_body(i, carry, strip_len=strip_len):
            block_step(i, strip_len)
            return carry

        jax.lax.fori_loop(p * _PHASE, (p + 1) * _PHASE, phase_body, 0,
                          unroll=False)


def kernel(boxes, scores):
    pad = _P - _N
    neg_scores = jnp.concatenate(
        [-scores, jnp.full((pad,), jnp.inf, dtype=jnp.float32)])
    boxes_p = jnp.concatenate(
        [boxes, jnp.zeros((pad, 4), dtype=jnp.float32)], axis=0)
    idx = jnp.arange(_P, dtype=jnp.int32)

    _, sx1, sy1, sx2, sy2, sidx = jax.lax.sort(
        (neg_scores, boxes_p[:, 0], boxes_p[:, 1], boxes_p[:, 2],
         boxes_p[:, 3], idx),
        num_keys=1, is_stable=True)

    comps_w = [c.reshape(1, _P) for c in (sx1, sy1, sx2, sy2)]

    keep_sorted = pl.pallas_call(
        _nms_kernel,
        out_shape=jax.ShapeDtypeStruct((_NB, _B), jnp.float32),
        scratch_shapes=[pltpu.VMEM((1, _P), jnp.float32),
                        pltpu.VMEM((1, _P), jnp.float32)],
    )(*comps_w)

    _, keep_f = jax.lax.sort((sidx, keep_sorted.reshape(_P)), num_keys=1,
                             is_stable=False)
    keep = (keep_f > 0.0)[:_N]
    masked = scores * keep.astype(scores.dtype)
    return masked, keep


# speedup vs baseline: 1.0975x; 1.0342x over previous
import jax
import jax.numpy as jnp
from jax.experimental import pallas as pl
from jax.experimental.pallas import tpu as pltpu

_N = 5000
_B = 128
_NB = 40
_P = _NB * _B
_PHASE = 2
_IOU_T = 0.5


def _nms_kernel(x1w, y1w, x2w, y2w, keep_out, arear, keeprow):
    keep_out[...] = jnp.zeros((_NB, _B), jnp.float32)
    keeprow[...] = jnp.zeros((1, _P), jnp.float32)
    arear[...] = (x2w[...] - x1w[...]) * (y2w[...] - y1w[...])

    rloc = jax.lax.broadcasted_iota(jnp.int32, (_B, _B), 0)
    cloc = jax.lax.broadcasted_iota(jnp.int32, (_B, _B), 1)

    def block_step(i, strip_len):
        bx1 = x1w[0:1, pl.ds(i * _B, _B)]
        by1 = y1w[0:1, pl.ds(i * _B, _B)]
        bx2 = x2w[0:1, pl.ds(i * _B, _B)]
        by2 = y2w[0:1, pl.ds(i * _B, _B)]
        barea = (bx2 - bx1) * (by2 - by1)
        rx1 = jnp.transpose(bx1, (1, 0))
        ry1 = jnp.transpose(by1, (1, 0))
        rx2 = jnp.transpose(bx2, (1, 0))
        ry2 = jnp.transpose(by2, (1, 0))
        rarea = jnp.transpose(barea, (1, 0))

        sx1 = x1w[0:1, 0:strip_len]
        sy1 = y1w[0:1, 0:strip_len]
        sx2 = x2w[0:1, 0:strip_len]
        sy2 = y2w[0:1, 0:strip_len]
        sarea = arear[0:1, 0:strip_len]
        jx1 = jnp.maximum(rx1, sx1)
        jy1 = jnp.maximum(ry1, sy1)
        jx2 = jnp.minimum(rx2, sx2)
        jy2 = jnp.minimum(ry2, sy2)
        jw = jnp.maximum(jx2 - jx1, 0.0)
        jh = jnp.maximum(jy2 - jy1, 0.0)
        jinter = jw * jh
        jiou = jinter / (sarea + rarea - jinter + 1e-9)
        over = jiou > _IOU_T

        kmask = jnp.where(over, keeprow[0:1, pl.ds(0, strip_len)], 0.0)
        sup_col = jnp.max(kmask, axis=1, keepdims=True)
        free = jnp.transpose((sup_col == 0.0).astype(jnp.float32), (1, 0))

        tx1 = jnp.maximum(rx1, bx1)
        ty1 = jnp.maximum(ry1, by1)
        tx2 = jnp.minimum(rx2, bx2)
        ty2 = jnp.minimum(ry2, by2)
        tw = jnp.maximum(tx2 - tx1, 0.0)
        th = jnp.maximum(ty2 - ty1, 0.0)
        tinter = tw * th
        tiou = tinter / (rarea + barea - tinter + 1e-9)
        tf = jnp.where((tiou > _IOU_T) & (rloc < cloc), 1.0, 0.0)

        def fcond(c):
            return c[2]

        def fbody(c):
            k, _, _ = c
            s = jnp.dot(k, tf, preferred_element_type=jnp.float32)
            k2 = free * (s == 0.0).astype(jnp.float32)
            return k2, k, jnp.any(k2 != k)

        k0 = free
        s0 = jnp.dot(k0, tf, preferred_element_type=jnp.float32)
        k1 = free * (s0 == 0.0).astype(jnp.float32)
        k, _, _ = jax.lax.while_loop(fcond, fbody, (k1, k0, jnp.any(k1 != k0)))

        keep_out[pl.ds(i, 1), :] = k
        keeprow[0:1, pl.ds(i * _B, _B)] = k

    for p in range(_NB // _PHASE):
        strip_len = (p + 1) * _PHASE * _B

        def phase_body(i, carry, strip_len=strip_len):
            block_step(i, strip_len)
            return carry

        jax.lax.fori_loop(p * _PHASE, (p + 1) * _PHASE, phase_body, 0,
                          unroll=False)


def kernel(boxes, scores):
    pad = _P - _N
    neg_scores = jnp.concatenate(
        [-scores, jnp.full((pad,), jnp.inf, dtype=jnp.float32)])
    boxes_p = jnp.concatenate(
        [boxes, jnp.zeros((pad, 4), dtype=jnp.float32)], axis=0)
    idx = jnp.arange(_P, dtype=jnp.int32)

    _, sx1, sy1, sx2, sy2, sidx = jax.lax.sort(
        (neg_scores, boxes_p[:, 0], boxes_p[:, 1], boxes_p[:, 2],
         boxes_p[:, 3], idx),
        num_keys=1, is_stable=True)

    comps_w = [c.reshape(1, _P) for c in (sx1, sy1, sx2, sy2)]

    keep_sorted = pl.pallas_call(
        _nms_kernel,
        out_shape=jax.ShapeDtypeStruct((_NB, _B), jnp.float32),
        scratch_shapes=[pltpu.VMEM((1, _P), jnp.float32),
                        pltpu.VMEM((1, _P), jnp.float32)],
    )(*comps_w)

    _, keep_f = jax.lax.sort((sidx, keep_sorted.reshape(_P)), num_keys=1,
                             is_stable=False)
    keep = (keep_f > 0.0)[:_N]
    masked = scores * keep.astype(scores.dtype)
    return masked, keep


# speedup vs baseline: 1.1043x; 1.0062x over previous
import jax
import jax.numpy as jnp
from jax.experimental import pallas as pl
from jax.experimental.pallas import tpu as pltpu

_N = 5000
_B = 128
_NB = 40
_P = _NB * _B
_PHASE = 1
_IOU_T = 0.5


def _nms_kernel(x1w, y1w, x2w, y2w, keep_out, arear, keeprow):
    keep_out[...] = jnp.zeros((_NB, _B), jnp.float32)
    keeprow[...] = jnp.zeros((1, _P), jnp.float32)
    arear[...] = (x2w[...] - x1w[...]) * (y2w[...] - y1w[...])

    rloc = jax.lax.broadcasted_iota(jnp.int32, (_B, _B), 0)
    cloc = jax.lax.broadcasted_iota(jnp.int32, (_B, _B), 1)

    def block_step(i, strip_len):
        bx1 = x1w[0:1, pl.ds(i * _B, _B)]
        by1 = y1w[0:1, pl.ds(i * _B, _B)]
        bx2 = x2w[0:1, pl.ds(i * _B, _B)]
        by2 = y2w[0:1, pl.ds(i * _B, _B)]
        barea = (bx2 - bx1) * (by2 - by1)
        rx1 = jnp.transpose(bx1, (1, 0))
        ry1 = jnp.transpose(by1, (1, 0))
        rx2 = jnp.transpose(bx2, (1, 0))
        ry2 = jnp.transpose(by2, (1, 0))
        rarea = jnp.transpose(barea, (1, 0))

        sx1 = x1w[0:1, 0:strip_len]
        sy1 = y1w[0:1, 0:strip_len]
        sx2 = x2w[0:1, 0:strip_len]
        sy2 = y2w[0:1, 0:strip_len]
        sarea = arear[0:1, 0:strip_len]
        jx1 = jnp.maximum(rx1, sx1)
        jy1 = jnp.maximum(ry1, sy1)
        jx2 = jnp.minimum(rx2, sx2)
        jy2 = jnp.minimum(ry2, sy2)
        jw = jnp.maximum(jx2 - jx1, 0.0)
        jh = jnp.maximum(jy2 - jy1, 0.0)
        jinter = jw * jh
        jiou = jinter / (sarea + rarea - jinter + 1e-9)
        over = jiou > _IOU_T

        kmask = jnp.where(over, keeprow[0:1, pl.ds(0, strip_len)], 0.0)
        sup_col = jnp.max(kmask, axis=1, keepdims=True)
        free = jnp.transpose((sup_col == 0.0).astype(jnp.float32), (1, 0))

        tx1 = jnp.maximum(rx1, bx1)
        ty1 = jnp.maximum(ry1, by1)
        tx2 = jnp.minimum(rx2, bx2)
        ty2 = jnp.minimum(ry2, by2)
        tw = jnp.maximum(tx2 - tx1, 0.0)
        th = jnp.maximum(ty2 - ty1, 0.0)
        tinter = tw * th
        tiou = tinter / (rarea + barea - tinter + 1e-9)
        tf = jnp.where((tiou > _IOU_T) & (rloc < cloc), 1.0, 0.0)

        def fcond(c):
            return c[2]

        def fbody(c):
            k, _, _ = c
            s = jnp.dot(k, tf, preferred_element_type=jnp.float32)
            k2 = free * (s == 0.0).astype(jnp.float32)
            return k2, k, jnp.any(k2 != k)

        k0 = free
        s0 = jnp.dot(k0, tf, preferred_element_type=jnp.float32)
        k1 = free * (s0 == 0.0).astype(jnp.float32)
        k, _, _ = jax.lax.while_loop(fcond, fbody, (k1, k0, jnp.any(k1 != k0)))

        keep_out[pl.ds(i, 1), :] = k
        keeprow[0:1, pl.ds(i * _B, _B)] = k

    for p in range(_NB // _PHASE):
        strip_len = (p + 1) * _PHASE * _B

        def phase_body(i, carry, strip_len=strip_len):
            block_step(i, strip_len)
            return carry

        jax.lax.fori_loop(p * _PHASE, (p + 1) * _PHASE, phase_body, 0,
                          unroll=False)


def kernel(boxes, scores):
    pad = _P - _N
    neg_scores = jnp.concatenate(
        [-scores, jnp.full((pad,), jnp.inf, dtype=jnp.float32)])
    boxes_p = jnp.concatenate(
        [boxes, jnp.zeros((pad, 4), dtype=jnp.float32)], axis=0)
    idx = jnp.arange(_P, dtype=jnp.int32)

    _, sx1, sy1, sx2, sy2, sidx = jax.lax.sort(
        (neg_scores, boxes_p[:, 0], boxes_p[:, 1], boxes_p[:, 2],
         boxes_p[:, 3], idx),
        num_keys=1, is_stable=True)

    comps_w = [c.reshape(1, _P) for c in (sx1, sy1, sx2, sy2)]

    keep_sorted = pl.pallas_call(
        _nms_kernel,
        out_shape=jax.ShapeDtypeStruct((_NB, _B), jnp.float32),
        scratch_shapes=[pltpu.VMEM((1, _P), jnp.float32),
                        pltpu.VMEM((1, _P), jnp.float32)],
    )(*comps_w)

    _, keep_f = jax.lax.sort((sidx, keep_sorted.reshape(_P)), num_keys=1,
                             is_stable=False)
    keep = (keep_f > 0.0)[:_N]
    masked = scores * keep.astype(scores.dtype)
    return masked, keep
